# Initial kernel scaffold; baseline (speedup 1.0000x reference)
import functools

import jax
import jax.numpy as jnp
from jax import lax
from jax.experimental import pallas as pl
from jax.experimental.pallas import tpu as pltpu
from jax.experimental.pallas import tpu_sc as plsc

_D = 128
_NC = 2
_NS = 16
_NW = _NC * _NS
_B = 128


def _pad_to(n, m):
    return ((n + m - 1) // m) * m


@functools.cache
def _make_deg_kernel(EP, NP):
    nb = EP // (_NW * _B)
    rpt = NP // _NS
    mesh = plsc.VectorSubcoreMesh(core_axis_name="c", subcore_axis_name="s")

    @functools.partial(
        pl.kernel,
        out_type=jax.ShapeDtypeStruct((_NC, NP), jnp.float32),
        mesh=mesh,
        scratch_types=[
            pltpu.VMEM((_B,), jnp.int32),
            pltpu.VMEM((_B,), jnp.float32),
            pltpu.VMEM((rpt,), jnp.float32),
            pltpu.VMEM_SHARED((NP,), jnp.float32),
        ],
    )
    def deg_kernel(dst_hbm, out_hbm, idx_v, ones_v, zeros_v, hist_s):
        cid = lax.axis_index("c")
        sid = lax.axis_index("s")
        wid = sid * _NC + cid
        one16 = jnp.ones((16,), jnp.float32)
        zero16 = jnp.zeros((16,), jnp.float32)
        for i in range(_B // 16):
            ones_v[pl.ds(i * 16, 16)] = one16

        def zbody(i, c):
            zeros_v[pl.ds(pl.multiple_of(i * 16, 8), 16)] = zero16
            return c
        lax.fori_loop(0, rpt // 16, zbody, 0)
        r0 = pl.multiple_of(sid * rpt, 8)
        pltpu.sync_copy(zeros_v, hist_s.at[pl.ds(r0, rpt)])
        plsc.subcore_barrier()

        ebase = wid * nb * _B

        def body(j, c):
            off = pl.multiple_of(ebase + j * _B, 8)
            pltpu.sync_copy(dst_hbm.at[pl.ds(off, _B)], idx_v)
            pltpu.sync_copy(ones_v, hist_s.at[idx_v], add=True)
            return c
        lax.fori_loop(0, nb, body, 0)
        plsc.subcore_barrier()
        pltpu.sync_copy(hist_s.at[pl.ds(r0, rpt)], out_hbm.at[cid, pl.ds(r0, rpt)])

    return deg_kernel


@functools.cache
def _make_scatter_kernel(EP, NP):
    nb = EP // (_NW * _B)
    rpt = NP // _NS
    mesh = plsc.VectorSubcoreMesh(core_axis_name="c", subcore_axis_name="s")

    @functools.partial(
        pl.kernel,
        out_type=jax.ShapeDtypeStruct((_NC, NP, _D), jnp.float32),
        mesh=mesh,
        scratch_types=[
            pltpu.VMEM((_B,), jnp.int32),
            pltpu.VMEM((_B,), jnp.int32),
            pltpu.VMEM((_B, _D), jnp.float32),
            pltpu.VMEM((8, _D), jnp.float32),
            pltpu.VMEM_SHARED((NP, _D), jnp.float32),
            pltpu.SemaphoreType.DMA,
        ],
    )
    def scatter_kernel(g_hbm, src_hbm, dst_hbm, out_hbm,
                       src_v, dst_v, rows_v, zeros_v, acc_s, sem):
        cid = lax.axis_index("c")
        sid = lax.axis_index("s")
        wid = sid * _NC + cid
        r0 = pl.multiple_of(sid * rpt, 8)

        @pl.when(cid == 0)
        def _():
            pltpu.sync_copy(g_hbm.at[pl.ds(r0, rpt)], acc_s.at[pl.ds(r0, rpt)])

        @pl.when(cid != 0)
        def _():
            zero16 = jnp.zeros((16,), jnp.float32)
            for i in range(8):
                for j in range(_D // 16):
                    zeros_v[i, pl.ds(j * 16, 16)] = zero16

            def zcopy(i, c):
                pltpu.sync_copy(
                    zeros_v, acc_s.at[pl.ds(pl.multiple_of(r0 + i * 8, 8), 8)])
                return c
            lax.fori_loop(0, rpt // 8, zcopy, 0)

        plsc.subcore_barrier()

        ebase = wid * nb * _B

        def body(j, c):
            off = pl.multiple_of(ebase + j * _B, 8)
            pltpu.sync_copy(src_hbm.at[pl.ds(off, _B)], src_v)
            pltpu.sync_copy(dst_hbm.at[pl.ds(off, _B)], dst_v)
            pltpu.async_copy(g_hbm.at[src_v], rows_v, sem).wait()
            pltpu.sync_copy(rows_v, acc_s.at[dst_v], add=True)
            return c
        lax.fori_loop(0, nb, body, 0)
        plsc.subcore_barrier()
        pltpu.sync_copy(acc_s.at[pl.ds(r0, rpt)], out_hbm.at[cid, pl.ds(r0, rpt)])

    return scatter_kernel


_BM = 1024


def _first_body(d0_ref, d1_ref, x_ref, w_ref, dinv_ref, g_ref):
    deg = d0_ref[...] + d1_ref[...] + 1.0
    dinv = lax.rsqrt(deg)
    dinv_ref[...] = dinv
    g_ref[...] = dinv * jnp.dot(x_ref[...], w_ref[...],
                                preferred_element_type=jnp.float32)


def _mid_body(p0_ref, p1_ref, dinv_ref, b_ref, w_ref, g_ref):
    dinv = dinv_ref[...]
    h = jnp.maximum(dinv * (p0_ref[...] + p1_ref[...]) + b_ref[...], 0.0)
    g_ref[...] = dinv * jnp.dot(h, w_ref[...],
                                preferred_element_type=jnp.float32)


def _last_body(p0_ref, p1_ref, dinv_ref, b_ref, o_ref):
    o_ref[...] = dinv_ref[...] * (p0_ref[...] + p1_ref[...]) + b_ref[...]


def _row_spec(i):
    return (i, 0)


def _const_spec(i):
    return (0, 0)


def _tc_first(d0, d1, x, w, NP):
    return pl.pallas_call(
        _first_body,
        grid=(NP // _BM,),
        in_specs=[pl.BlockSpec((_BM, 1), _row_spec),
                  pl.BlockSpec((_BM, 1), _row_spec),
                  pl.BlockSpec((_BM, _D), _row_spec),
                  pl.BlockSpec((_D, _D), _const_spec)],
        out_specs=[pl.BlockSpec((_BM, 1), _row_spec),
                   pl.BlockSpec((_BM, _D), _row_spec)],
        out_shape=[jax.ShapeDtypeStruct((NP, 1), jnp.float32),
                   jax.ShapeDtypeStruct((NP, _D), jnp.float32)],
    )(d0, d1, x, w)


def _tc_mid(p0, p1, dinv, b, w, NP):
    return pl.pallas_call(
        _mid_body,
        grid=(NP // _BM,),
        in_specs=[pl.BlockSpec((_BM, _D), _row_spec),
                  pl.BlockSpec((_BM, _D), _row_spec),
                  pl.BlockSpec((_BM, 1), _row_spec),
                  pl.BlockSpec((1, _D), _const_spec),
                  pl.BlockSpec((_D, _D), _const_spec)],
        out_specs=pl.BlockSpec((_BM, _D), _row_spec),
        out_shape=jax.ShapeDtypeStruct((NP, _D), jnp.float32),
    )(p0, p1, dinv, b, w)


def _tc_last(p0, p1, dinv, b, NP):
    return pl.pallas_call(
        _last_body,
        grid=(NP // _BM,),
        in_specs=[pl.BlockSpec((_BM, _D), _row_spec),
                  pl.BlockSpec((_BM, _D), _row_spec),
                  pl.BlockSpec((_BM, 1), _row_spec),
                  pl.BlockSpec((1, _D), _const_spec)],
        out_specs=pl.BlockSpec((_BM, _D), _row_spec),
        out_shape=jax.ShapeDtypeStruct((NP, _D), jnp.float32),
    )(p0, p1, dinv, b)


def kernel(x, edge_index, W0, b0, W1, b1, W2, b2):
    N, D = x.shape
    E = edge_index.shape[1]
    NP = _pad_to(N, _BM)
    EP = _pad_to(E, _NW * _B)

    xp = jnp.pad(x, ((0, NP - N), (0, 0)))
    src = jnp.pad(edge_index[0], (0, EP - E))
    dst = jnp.pad(edge_index[1], (0, EP - E), constant_values=N)

    deg_kernel = _make_deg_kernel(EP, NP)
    scatter_kernel = _make_scatter_kernel(EP, NP)

    degp = deg_kernel(dst)
    d0 = degp[0].reshape(NP, 1)
    d1 = degp[1].reshape(NP, 1)

    dinv, g = _tc_first(d0, d1, xp, W0, NP)
    b0r = b0.reshape(1, _D)
    b1r = b1.reshape(1, _D)
    b2r = b2.reshape(1, _D)

    p = scatter_kernel(g, src, dst)
    g = _tc_mid(p[0], p[1], dinv, b0r, W1, NP)

    p = scatter_kernel(g, src, dst)
    g = _tc_mid(p[0], p[1], dinv, b1r, W2, NP)

    p = scatter_kernel(g, src, dst)
    out = _tc_last(p[0], p[1], dinv, b2r, NP)
    return out[:N]


# speedup vs baseline: 9.0954x; 9.0954x over previous
import functools

import jax
import jax.numpy as jnp
from jax import lax
from jax.experimental import pallas as pl
from jax.experimental.pallas import tpu as pltpu
from jax.experimental.pallas import tpu_sc as plsc

_D = 128
_NC = 2
_NS = 16
_NW = _NC * _NS
_B = 128


def _pad_to(n, m):
    return ((n + m - 1) // m) * m


@functools.cache
def _make_deg_kernel(EP, NP):
    nb = EP // (_NW * _B)
    rpt = NP // _NS
    mesh = plsc.VectorSubcoreMesh(core_axis_name="c", subcore_axis_name="s",
                                  num_cores=_NC, num_subcores=_NS)

    @functools.partial(
        pl.kernel,
        out_type=jax.ShapeDtypeStruct((_NC, NP), jnp.float32),
        mesh=mesh,
        scratch_types=[
            pltpu.VMEM((_B,), jnp.int32),
            pltpu.VMEM((_B,), jnp.float32),
            pltpu.VMEM((rpt,), jnp.float32),
            pltpu.VMEM_SHARED((NP,), jnp.float32),
        ],
    )
    def deg_kernel(dst_hbm, out_hbm, idx_v, ones_v, zeros_v, hist_s):
        cid = lax.axis_index("c")
        sid = lax.axis_index("s")
        wid = sid * _NC + cid
        one16 = jnp.ones((16,), jnp.float32)
        zero16 = jnp.zeros((16,), jnp.float32)
        for i in range(_B // 16):
            ones_v[pl.ds(i * 16, 16)] = one16

        def zbody(i, c):
            zeros_v[pl.ds(pl.multiple_of(i * 16, 8), 16)] = zero16
            return c
        lax.fori_loop(0, rpt // 16, zbody, 0)
        r0 = pl.multiple_of(sid * rpt, 8)
        pltpu.sync_copy(zeros_v, hist_s.at[pl.ds(r0, rpt)])
        plsc.subcore_barrier()

        ebase = wid * nb * _B

        def body(j, c):
            off = pl.multiple_of(ebase + j * _B, 8)
            pltpu.sync_copy(dst_hbm.at[pl.ds(off, _B)], idx_v)
            pltpu.sync_copy(ones_v, hist_s.at[idx_v], add=True)
            return c
        lax.fori_loop(0, nb, body, 0)
        plsc.subcore_barrier()
        pltpu.sync_copy(hist_s.at[pl.ds(r0, rpt)], out_hbm.at[cid, pl.ds(r0, rpt)])

    return deg_kernel


@functools.cache
def _make_scatter_kernel(EP, NP):
    nb = EP // (_NW * _B)
    rpt = NP // _NS
    mesh = plsc.VectorSubcoreMesh(core_axis_name="c", subcore_axis_name="s",
                                  num_cores=_NC, num_subcores=_NS)

    @functools.partial(
        pl.kernel,
        out_type=jax.ShapeDtypeStruct((_NC, NP, _D), jnp.float32),
        mesh=mesh,
        scratch_types=[
            pltpu.VMEM((_B,), jnp.int32),
            pltpu.VMEM((_B,), jnp.int32),
            pltpu.VMEM((_B, _D), jnp.float32),
            pltpu.VMEM((8, _D), jnp.float32),
            pltpu.VMEM_SHARED((NP, _D), jnp.float32),
            pltpu.SemaphoreType.DMA,
        ],
    )
    def scatter_kernel(g_hbm, src_hbm, dst_hbm, out_hbm,
                       src_v, dst_v, rows_v, zeros_v, acc_s, sem):
        cid = lax.axis_index("c")
        sid = lax.axis_index("s")
        wid = sid * _NC + cid
        r0 = pl.multiple_of(sid * rpt, 8)

        @pl.when(cid == 0)
        def _():
            pltpu.sync_copy(g_hbm.at[pl.ds(r0, rpt)], acc_s.at[pl.ds(r0, rpt)])

        @pl.when(cid != 0)
        def _():
            zero16 = jnp.zeros((16,), jnp.float32)
            for i in range(8):
                for j in range(_D // 16):
                    zeros_v[i, pl.ds(j * 16, 16)] = zero16

            def zcopy(i, c):
                pltpu.sync_copy(
                    zeros_v, acc_s.at[pl.ds(pl.multiple_of(r0 + i * 8, 8), 8)])
                return c
            lax.fori_loop(0, rpt // 8, zcopy, 0)

        plsc.subcore_barrier()

        ebase = wid * nb * _B

        def body(j, c):
            off = pl.multiple_of(ebase + j * _B, 8)
            pltpu.sync_copy(src_hbm.at[pl.ds(off, _B)], src_v)
            pltpu.sync_copy(dst_hbm.at[pl.ds(off, _B)], dst_v)
            pltpu.async_copy(g_hbm.at[src_v], rows_v, sem).wait()
            pltpu.sync_copy(rows_v, acc_s.at[dst_v], add=True)
            return c
        lax.fori_loop(0, nb, body, 0)
        plsc.subcore_barrier()
        pltpu.sync_copy(acc_s.at[pl.ds(r0, rpt)], out_hbm.at[cid, pl.ds(r0, rpt)])

    return scatter_kernel


_BM = 1024


def _first_body(d0_ref, d1_ref, x_ref, w_ref, dinv_ref, g_ref):
    deg = d0_ref[...] + d1_ref[...] + 1.0
    dinv = lax.rsqrt(deg)
    dinv_ref[...] = dinv
    g_ref[...] = dinv * jnp.dot(x_ref[...], w_ref[...],
                                preferred_element_type=jnp.float32)


def _mid_body(p0_ref, p1_ref, dinv_ref, b_ref, w_ref, g_ref):
    dinv = dinv_ref[...]
    h = jnp.maximum(dinv * (p0_ref[...] + p1_ref[...]) + b_ref[...], 0.0)
    g_ref[...] = dinv * jnp.dot(h, w_ref[...],
                                preferred_element_type=jnp.float32)


def _last_body(p0_ref, p1_ref, dinv_ref, b_ref, o_ref):
    o_ref[...] = dinv_ref[...] * (p0_ref[...] + p1_ref[...]) + b_ref[...]


def _row_spec(i):
    return (i, 0)


def _const_spec(i):
    return (0, 0)


def _tc_first(d0, d1, x, w, NP):
    return pl.pallas_call(
        _first_body,
        grid=(NP // _BM,),
        in_specs=[pl.BlockSpec((_BM, 1), _row_spec),
                  pl.BlockSpec((_BM, 1), _row_spec),
                  pl.BlockSpec((_BM, _D), _row_spec),
                  pl.BlockSpec((_D, _D), _const_spec)],
        out_specs=[pl.BlockSpec((_BM, 1), _row_spec),
                   pl.BlockSpec((_BM, _D), _row_spec)],
        out_shape=[jax.ShapeDtypeStruct((NP, 1), jnp.float32),
                   jax.ShapeDtypeStruct((NP, _D), jnp.float32)],
    )(d0, d1, x, w)


def _tc_mid(p0, p1, dinv, b, w, NP):
    return pl.pallas_call(
        _mid_body,
        grid=(NP // _BM,),
        in_specs=[pl.BlockSpec((_BM, _D), _row_spec),
                  pl.BlockSpec((_BM, _D), _row_spec),
                  pl.BlockSpec((_BM, 1), _row_spec),
                  pl.BlockSpec((1, _D), _const_spec),
                  pl.BlockSpec((_D, _D), _const_spec)],
        out_specs=pl.BlockSpec((_BM, _D), _row_spec),
        out_shape=jax.ShapeDtypeStruct((NP, _D), jnp.float32),
    )(p0, p1, dinv, b, w)


def _tc_last(p0, p1, dinv, b, NP):
    return pl.pallas_call(
        _last_body,
        grid=(NP // _BM,),
        in_specs=[pl.BlockSpec((_BM, _D), _row_spec),
                  pl.BlockSpec((_BM, _D), _row_spec),
                  pl.BlockSpec((_BM, 1), _row_spec),
                  pl.BlockSpec((1, _D), _const_spec)],
        out_specs=pl.BlockSpec((_BM, _D), _row_spec),
        out_shape=jax.ShapeDtypeStruct((NP, _D), jnp.float32),
    )(p0, p1, dinv, b)


def kernel(x, edge_index, W0, b0, W1, b1, W2, b2):
    N, D = x.shape
    E = edge_index.shape[1]
    NP = _pad_to(N, _BM)
    EP = _pad_to(E, _NW * _B)

    xp = jnp.pad(x, ((0, NP - N), (0, 0)))
    src = jnp.pad(edge_index[0], (0, EP - E))
    dst = jnp.pad(edge_index[1], (0, EP - E), constant_values=N)

    deg_kernel = _make_deg_kernel(EP, NP)
    scatter_kernel = _make_scatter_kernel(EP, NP)

    degp = deg_kernel(dst)
    d0 = degp[0].reshape(NP, 1)
    d1 = degp[1].reshape(NP, 1)

    dinv, g = _tc_first(d0, d1, xp, W0, NP)
    b0r = b0.reshape(1, _D)
    b1r = b1.reshape(1, _D)
    b2r = b2.reshape(1, _D)

    p = scatter_kernel(g, src, dst)
    g = _tc_mid(p[0], p[1], dinv, b0r, W1, NP)

    p = scatter_kernel(g, src, dst)
    g = _tc_mid(p[0], p[1], dinv, b1r, W2, NP)

    p = scatter_kernel(g, src, dst)
    out = _tc_last(p[0], p[1], dinv, b2r, NP)
    return out[:N]
